# Initial kernel scaffold; baseline (speedup 1.0000x reference)
import functools

import jax
import jax.numpy as jnp
from jax import lax
from jax.experimental import pallas as pl
from jax.experimental.pallas import tpu as pltpu
from jax.experimental.pallas import tpu_sc as plsc

NUM_ROWS = 4096 * 200
DIM = 32
CHUNK = 128
NBUF = 8

_INFO = plsc.get_sparse_core_info()
_NC = _INFO.num_cores
_NS = _INFO.num_subcores
NW = _NC * _NS
CHUNKS_TOTAL = NUM_ROWS // CHUNK
CHUNKS_PER_W = CHUNKS_TOTAL // NW
GROUPS = CHUNKS_PER_W // NBUF


def _embed_body(idx_hbm, table_hbm, out_hbm, idx_v, rows_v, gsem, osem):
  wid = lax.axis_index("s") * _NC + lax.axis_index("c")
  row0 = wid * CHUNKS_PER_W

  pltpu.sync_copy(idx_hbm.at[pl.ds(row0, CHUNKS_PER_W)], idx_v)

  def gather(j, b):
    return pltpu.make_async_copy(table_hbm.at[idx_v.at[j]], rows_v.at[b],
                                 gsem.at[b])

  def outcopy(j, b):
    return pltpu.make_async_copy(
        rows_v.at[b], out_hbm.at[pl.ds((row0 + j) * CHUNK, CHUNK)],
        osem.at[b])

  for b in range(NBUF):
    gather(b, b).start()

  def body(g, carry):
    j0 = g * NBUF
    for b in range(NBUF):
      gather(j0 + b, b).wait()
      outcopy(j0 + b, b).start()
    for b in range(NBUF):
      outcopy(j0 + b, b).wait()
      gather(j0 + NBUF + b, b).start()
    return carry

  lax.fori_loop(0, GROUPS - 1, body, 0)

  j0 = (GROUPS - 1) * NBUF
  for b in range(NBUF):
    gather(j0 + b, b).wait()
    outcopy(j0 + b, b).start()
  for b in range(NBUF):
    outcopy(j0 + b, b).wait()


@jax.jit
def _embed(x_flat2d, weight):
  mesh = plsc.VectorSubcoreMesh(core_axis_name="c", subcore_axis_name="s")
  run = pl.kernel(
      _embed_body,
      out_type=jax.ShapeDtypeStruct((NUM_ROWS, DIM), jnp.float32),
      mesh=mesh,
      scratch_types=[
          pltpu.VMEM((CHUNKS_PER_W, CHUNK), jnp.int32),
          pltpu.VMEM((NBUF, CHUNK, DIM), jnp.float32),
          pltpu.SemaphoreType.DMA((NBUF,)),
          pltpu.SemaphoreType.DMA((NBUF,)),
      ],
  )
  return run(x_flat2d, weight)


def kernel(x, weight):
  x_flat2d = x.reshape(CHUNKS_TOTAL, CHUNK).astype(jnp.int32)
  out = _embed(x_flat2d, weight)
  return out.reshape(x.shape + (DIM,))


# speedup vs baseline: 1.4985x; 1.4985x over previous
import functools

import jax
import jax.numpy as jnp
from jax import lax
from jax.experimental import pallas as pl
from jax.experimental.pallas import tpu as pltpu
from jax.experimental.pallas import tpu_sc as plsc

NUM_ROWS = 4096 * 200
DIM = 32
CHUNK = 128
NBUF = 8

_INFO = plsc.get_sparse_core_info()
_NC = _INFO.num_cores
_NS = _INFO.num_subcores
NW = _NC * _NS
CHUNKS_TOTAL = NUM_ROWS // CHUNK
CHUNKS_PER_W = CHUNKS_TOTAL // NW
GROUPS = CHUNKS_PER_W // NBUF


def _embed_body(idx_hbm, table_hbm, out_hbm, idx_v, rows_v, gsem, osem):
  wid = lax.axis_index("s") * _NC + lax.axis_index("c")
  row0 = wid * CHUNKS_PER_W

  pltpu.sync_copy(idx_hbm.at[pl.ds(row0, CHUNKS_PER_W)], idx_v)

  def gather(j, b):
    return pltpu.make_async_copy(table_hbm.at[idx_v.at[j]], rows_v.at[b],
                                 gsem.at[b])

  def outcopy(j, b):
    return pltpu.make_async_copy(
        rows_v.at[b], out_hbm.at[pl.ds((row0 + j) * CHUNK, CHUNK)],
        osem.at[b])

  for b in range(NBUF):
    gather(b, b).start()

  def body(g, carry):
    j0 = g * NBUF
    for b in range(NBUF):
      gather(j0 + b, b).wait()
      outcopy(j0 + b, b).start()
    for b in range(NBUF):
      outcopy(j0 + b, b).wait()
      gather(j0 + NBUF + b, b).start()
    return carry

  lax.fori_loop(0, GROUPS - 1, body, 0)

  j0 = (GROUPS - 1) * NBUF
  for b in range(NBUF):
    gather(j0 + b, b).wait()
    outcopy(j0 + b, b).start()
  for b in range(NBUF):
    outcopy(j0 + b, b).wait()


@jax.jit
def _embed(x_flat2d, weight):
  mesh = plsc.VectorSubcoreMesh(core_axis_name="c", subcore_axis_name="s")
  run = pl.kernel(
      _embed_body,
      out_type=jax.ShapeDtypeStruct((NUM_ROWS, DIM), jnp.float32),
      mesh=mesh,
      scratch_types=[
          pltpu.VMEM((CHUNKS_PER_W, CHUNK), jnp.int32),
          pltpu.VMEM((NBUF, CHUNK, DIM), jnp.float32),
          pltpu.SemaphoreType.DMA((NBUF,)),
          pltpu.SemaphoreType.DMA((NBUF,)),
      ],
      compiler_params=pltpu.CompilerParams(use_tc_tiling_on_sc=False),
  )
  return run(x_flat2d, weight)


def kernel(x, weight):
  x_flat2d = x.reshape(CHUNKS_TOTAL, CHUNK).astype(jnp.int32)
  out = _embed(x_flat2d, weight)
  return out.reshape(x.shape + (DIM,))
